# baseline (speedup 1.0000x reference)
import jax
import jax.numpy as jnp
from jax import lax
from jax.experimental import pallas as pl
from jax.experimental.pallas import tpu as pltpu

_CH = 25
_B = 64
_C = 10647
_BC = 2688
_NCB = 4


def _fold8(t):
    parts = [t[i * 8:(i + 1) * 8] for i in range(8)]
    while len(parts) > 1:
        parts = [a + b for a, b in zip(parts[::2], parts[1::2])]
    return parts[0]


def _tc_body(x_ref, y_ref, o_ref, acc_ref, m_ref):
    ch = pl.program_id(0)
    cb = pl.program_id(1)

    xb = x_ref[0]
    yb = y_ref[0]

    @pl.when(ch == 0)
    def _():
        m_ref[cb] = yb

    col_ok = (cb * _BC + lax.broadcasted_iota(jnp.int32, (_B, _BC), 1)) < _C
    mb = m_ref[cb] > 0.5
    mm = mb & col_ok
    d = yb - xb
    sq = d * d

    @pl.when(ch == 0)
    def _():
        t = jnp.where(mm, sq, 0.0) + jnp.where(
            col_ok & jnp.logical_not(mb), 0.1 * (xb * xb), 0.0)

        @pl.when(cb == 0)
        def _():
            acc_ref[...] = _fold8(t)

        @pl.when(cb > 0)
        def _():
            acc_ref[...] += _fold8(t)

    @pl.when(ch > 0)
    def _():
        acc_ref[...] += _fold8(jnp.where(mm, sq, 0.0))

    @pl.when((ch == _CH - 1) & (cb == _NCB - 1))
    def _():
        o_ref[0] = jnp.sum(acc_ref[...])


_tc_call = pl.pallas_call(
    _tc_body,
    grid=(_CH, _NCB),
    in_specs=[
        pl.BlockSpec((1, _B, _BC), lambda ch, cb: (ch, 0, cb)),
        pl.BlockSpec((1, _B, _BC), lambda ch, cb: (ch, 0, cb)),
    ],
    out_specs=pl.BlockSpec(memory_space=pltpu.SMEM),
    out_shape=jax.ShapeDtypeStruct((1,), jnp.float32),
    scratch_shapes=[
        pltpu.VMEM((8, _BC), jnp.float32),
        pltpu.VMEM((_NCB, _B, _BC), jnp.float32),
    ],
)


def kernel(x, y):
    xt = jnp.transpose(x, (2, 0, 1))
    yt = jnp.transpose(y, (2, 0, 1))
    out = _tc_call(xt, yt)
    return out[0]


# speedup vs baseline: 1.8167x; 1.8167x over previous
import jax
import jax.numpy as jnp
from jax import lax
from jax.experimental import pallas as pl
from jax.experimental.pallas import tpu as pltpu

_CH = 25
_B = 64
_C = 10647


def _fold8(t):
    parts = [t[i * 8:(i + 1) * 8] for i in range(8)]
    while len(parts) > 1:
        parts = [a + b for a, b in zip(parts[::2], parts[1::2])]
    return parts[0]


def _tc_body(x_ref, y_ref, o_ref, acc_ref, m_ref):
    ch = pl.program_id(0)

    xb = x_ref[0]
    yb = y_ref[0]

    @pl.when(ch == 0)
    def _():
        m_ref[...] = jnp.where(yb > 0.5, 1.0, 0.0).astype(jnp.float32)

    m01 = m_ref[...]
    d = yb - xb
    sq = d * d

    @pl.when(ch == 0)
    def _():
        acc_ref[...] = _fold8(sq * m01 + (0.1 * (xb * xb)) * (1.0 - m01))

    @pl.when(ch > 0)
    def _():
        acc_ref[...] += _fold8(sq * m01)

    @pl.when(ch == _CH - 1)
    def _():
        o_ref[0] = jnp.sum(acc_ref[...])


_tc_call = pl.pallas_call(
    _tc_body,
    grid=(_CH,),
    in_specs=[
        pl.BlockSpec((1, _B, _C), lambda ch: (ch, 0, 0)),
        pl.BlockSpec((1, _B, _C), lambda ch: (ch, 0, 0)),
    ],
    out_specs=pl.BlockSpec(memory_space=pltpu.SMEM),
    out_shape=jax.ShapeDtypeStruct((1,), jnp.float32),
    scratch_shapes=[
        pltpu.VMEM((8, _C), jnp.float32),
        pltpu.VMEM((_B, _C), jnp.float32),
    ],
)


def kernel(x, y):
    xt = jnp.transpose(x, (2, 0, 1))
    yt = jnp.transpose(y, (2, 0, 1))
    out = _tc_call(xt, yt)
    return out[0]
